# Initial kernel scaffold; baseline (speedup 1.0000x reference)
import jax
import jax.numpy as jnp
from jax.experimental import pallas as pl


def kernel(sentence_tensor, sentence_indices, W, b):
    raise NotImplementedError("write your pallas kernel here")


# speedup vs baseline: 77.3198x; 77.3198x over previous
import functools

import jax
import jax.numpy as jnp
from jax.experimental import pallas as pl


def _batch_body(st_ref, starts_ref, ends_ref, w_ref, out_ref):
    st = st_ref[0]
    T, D = st.shape
    logits = jnp.dot(st, w_ref[:, 0:1], preferred_element_type=jnp.float32)
    m = jnp.max(logits)
    e = jnp.exp(logits - m)
    starts = starts_ref[0, 0]
    ends = ends_ref[0, 0]
    S = starts.shape[0]
    t_idx = jax.lax.broadcasted_iota(jnp.int32, (S, T), 1)
    in_span = (t_idx >= starts[:, None]) & (t_idx <= ends[:, None])
    a = jnp.where(in_span, e[:, 0][None, :], 0.0)
    denom = jnp.sum(a, axis=1, keepdims=True)
    a = a / denom
    out_ref[0] = jnp.dot(a, st, preferred_element_type=jnp.float32)


def kernel(sentence_tensor, sentence_indices, W, b):
    B, T, D = sentence_tensor.shape
    S = sentence_indices.shape[1]
    starts = sentence_indices[..., 0].reshape(B, 1, S).astype(jnp.int32)
    ends = sentence_indices[..., 1].reshape(B, 1, S).astype(jnp.int32)
    out = pl.pallas_call(
        _batch_body,
        grid=(B,),
        in_specs=[
            pl.BlockSpec((1, T, D), lambda i: (i, 0, 0)),
            pl.BlockSpec((1, 1, S), lambda i: (i, 0, 0)),
            pl.BlockSpec((1, 1, S), lambda i: (i, 0, 0)),
            pl.BlockSpec((D, 1), lambda i: (0, 0)),
        ],
        out_specs=pl.BlockSpec((1, S, D), lambda i: (i, 0, 0)),
        out_shape=jax.ShapeDtypeStruct((B, S, D), jnp.float32),
    )(sentence_tensor, starts, ends, W)
    return out
